# baseline (speedup 1.0000x reference)
import functools

import jax
import jax.numpy as jnp
from jax import lax
from jax.experimental import pallas as pl
from jax.experimental.pallas import tpu as pltpu
from jax.experimental.pallas import tpu_sc as plsc

_D = 200
_D0 = 128
_D1 = 72
_S = 50
_SP = 56
_NC = 2
_NS = 16
_NW = _NC * _NS
_NBUF = 4


@functools.cache
def _make_sc_gather(nb: int):
    b_per_w = nb // _NW
    mesh = plsc.VectorSubcoreMesh(core_axis_name="c", subcore_axis_name="s")

    @functools.partial(
        pl.kernel,
        mesh=mesh,
        out_type=jax.ShapeDtypeStruct((nb, _S, _D), jnp.float32),
        scratch_types=[
            pltpu.VMEM((b_per_w * _SP,), jnp.int32),
            [pltpu.VMEM((_S, _D0), jnp.float32) for _ in range(_NBUF)],
            [pltpu.VMEM((_S, _D0), jnp.float32) for _ in range(_NBUF)],
            [pltpu.VMEM((_S, _D1), jnp.float32) for _ in range(_NBUF)],
            [pltpu.SemaphoreType.DMA for _ in range(_NBUF)],
            [pltpu.SemaphoreType.DMA for _ in range(_NBUF)],
            [pltpu.SemaphoreType.DMA for _ in range(_NBUF)],
        ],
    )
    def gather_kernel(idx_hbm, table_hbm, out_hbm, idx_v,
                      bufs0, bufs1, tails, sems0, sems1, ssems):
        wid = lax.axis_index("s") * _NC + lax.axis_index("c")
        base = wid * b_per_w
        t0 = table_hbm.at[:, pl.ds(0, _D0)]
        t1_hbm = table_hbm.at[:, pl.ds(_D0, _D0)]

        pltpu.sync_copy(idx_hbm.at[pl.ds(base * _SP, b_per_w * _SP)], idx_v)

        def gather_start(i, k):
            ids = idx_v.at[pl.ds(i * _SP, _S)]
            pltpu.async_copy(t0.at[ids], bufs0[k], sems0[k])
            pltpu.async_copy(t1_hbm.at[ids], bufs1[k], sems1[k])

        def gather_wait(k):
            ids = idx_v.at[pl.ds(0, _S)]
            pltpu.make_async_copy(t0.at[ids], bufs0[k], sems0[k]).wait()
            pltpu.make_async_copy(t1_hbm.at[ids], bufs1[k], sems1[k]).wait()

        def store_wait(k):
            pltpu.make_async_copy(
                bufs0[k], out_hbm.at[base, :, pl.ds(0, _D0)], ssems[k]).wait()
            pltpu.make_async_copy(
                tails[k], out_hbm.at[base, :, pl.ds(_D0, _D1)],
                ssems[k]).wait()

        for k in range(2):
            gather_start(k, k)

        def group(g, carry):
            for j in range(_NBUF):
                i = g * _NBUF + j
                b = base + i
                gather_wait(j)
                pltpu.async_copy(
                    bufs0[j], out_hbm.at[b, :, pl.ds(0, _D0)], ssems[j])
                for r in range(_S):
                    for o in (0, 16, 32, 48, 56):
                        tails[j][r, pl.ds(o, 16)] = bufs1[j][r, pl.ds(o, 16)]
                pltpu.async_copy(
                    tails[j], out_hbm.at[b, :, pl.ds(_D0, _D1)], ssems[j])
                k2 = (j + 2) % _NBUF

                @pl.when(i >= 2)
                def _():
                    store_wait(k2)

                nxt = i + 2

                @pl.when(nxt < b_per_w)
                def _():
                    gather_start(nxt, k2)
            return carry

        lax.fori_loop(0, b_per_w // _NBUF, group, 0, unroll=False)
        store_wait((b_per_w - 2) % _NBUF)
        store_wait((b_per_w - 1) % _NBUF)

    return gather_kernel


def kernel(input, table):
    nb = input.shape[0]
    idx1 = jnp.pad(input, ((0, 0), (0, _SP - _S))).reshape(-1)
    tp = jnp.pad(table, ((0, 0), (0, 2 * _D0 - _D)))
    return _make_sc_gather(nb)(idx1, tp)


# speedup vs baseline: 1.5821x; 1.5821x over previous
import functools

import jax
import jax.numpy as jnp
from jax import lax
from jax.experimental import pallas as pl
from jax.experimental.pallas import tpu as pltpu
from jax.experimental.pallas import tpu_sc as plsc

_D = 200
_D0 = 128
_D1 = 72
_S = 50
_SP = 56
_NC = 2
_NS = 16
_NW = _NC * _NS
_NBUF = 4


@functools.cache
def _make_sc_gather(nb: int):
    b_per_w = nb // _NW
    mesh = plsc.VectorSubcoreMesh(core_axis_name="c", subcore_axis_name="s")

    @functools.partial(
        pl.kernel,
        mesh=mesh,
        out_type=jax.ShapeDtypeStruct((nb, _S, _D), jnp.float32),
        scratch_types=[
            pltpu.VMEM((b_per_w * _SP,), jnp.int32),
            [pltpu.VMEM((_S, _D0), jnp.float32) for _ in range(_NBUF)],
            [pltpu.VMEM((_S, _D0), jnp.float32) for _ in range(_NBUF)],
            [pltpu.VMEM((_S, _D1), jnp.float32) for _ in range(_NBUF)],
            [pltpu.SemaphoreType.DMA for _ in range(_NBUF)],
            [pltpu.SemaphoreType.DMA for _ in range(_NBUF)],
            [pltpu.SemaphoreType.DMA for _ in range(_NBUF)],
        ],
    )
    def gather_kernel(idx_hbm, t0, t1_hbm, out_hbm, idx_v,
                      bufs0, bufs1, tails, sems0, sems1, ssems):
        wid = lax.axis_index("s") * _NC + lax.axis_index("c")
        base = wid * b_per_w

        pltpu.sync_copy(idx_hbm.at[pl.ds(base * _SP, b_per_w * _SP)], idx_v)

        def gather_start(i, k):
            ids = idx_v.at[pl.ds(i * _SP, _S)]
            pltpu.async_copy(t0.at[ids], bufs0[k], sems0[k])
            pltpu.async_copy(t1_hbm.at[ids], bufs1[k], sems1[k])

        def gather_wait(k):
            ids = idx_v.at[pl.ds(0, _S)]
            pltpu.make_async_copy(t0.at[ids], bufs0[k], sems0[k]).wait()
            pltpu.make_async_copy(t1_hbm.at[ids], bufs1[k], sems1[k]).wait()

        def store_wait(k):
            pltpu.make_async_copy(
                bufs0[k], out_hbm.at[base, :, pl.ds(0, _D0)], ssems[k]).wait()
            pltpu.make_async_copy(
                tails[k], out_hbm.at[base, :, pl.ds(_D0, _D1)],
                ssems[k]).wait()

        for k in range(2):
            gather_start(k, k)

        def group(g, carry):
            for j in range(_NBUF):
                i = g * _NBUF + j
                b = base + i
                gather_wait(j)
                pltpu.async_copy(
                    bufs0[j], out_hbm.at[b, :, pl.ds(0, _D0)], ssems[j])
                for r in range(_S):
                    for o in (0, 16, 32, 48, 56):
                        tails[j][r, pl.ds(o, 16)] = bufs1[j][r, pl.ds(o, 16)]
                pltpu.async_copy(
                    tails[j], out_hbm.at[b, :, pl.ds(_D0, _D1)], ssems[j])
                k2 = (j + 2) % _NBUF

                @pl.when(i >= 2)
                def _():
                    store_wait(k2)

                nxt = i + 2

                @pl.when(nxt < b_per_w)
                def _():
                    gather_start(nxt, k2)
            return carry

        lax.fori_loop(0, b_per_w // _NBUF, group, 0, unroll=False)
        store_wait((b_per_w - 2) % _NBUF)
        store_wait((b_per_w - 1) % _NBUF)

    return gather_kernel


def kernel(input, table):
    nb = input.shape[0]
    idx1 = jnp.pad(input, ((0, 0), (0, _SP - _S))).reshape(-1)
    t0 = lax.slice(table, (0, 0), (table.shape[0], _D0))
    t1 = jnp.pad(table[:, _D0:], ((0, 0), (0, _D0 - _D1)))
    return _make_sc_gather(nb)(idx1, t0, t1)


# speedup vs baseline: 1.6571x; 1.0474x over previous
import functools

import jax
import jax.numpy as jnp
from jax import lax
from jax.experimental import pallas as pl
from jax.experimental.pallas import tpu as pltpu
from jax.experimental.pallas import tpu_sc as plsc

_D = 200
_D0 = 128
_D1 = 72
_S = 50
_SP = 56
_NC = 2
_NS = 16
_NW = _NC * _NS
_NBUF = 4


@functools.cache
def _make_sc_gather(nb: int):
    b_per_w = nb // _NW
    mesh = plsc.VectorSubcoreMesh(core_axis_name="c", subcore_axis_name="s")

    @functools.partial(
        pl.kernel,
        mesh=mesh,
        out_type=jax.ShapeDtypeStruct((nb, _S, _D), jnp.float32),
        scratch_types=[
            pltpu.VMEM((b_per_w * _SP,), jnp.int32),
            [pltpu.VMEM((_S, _D0), jnp.float32) for _ in range(_NBUF)],
            [pltpu.VMEM((_S, _D0), jnp.float32) for _ in range(_NBUF)],
            [pltpu.VMEM((_S, _D1), jnp.float32) for _ in range(_NBUF)],
            [pltpu.SemaphoreType.DMA for _ in range(_NBUF)],
            [pltpu.SemaphoreType.DMA for _ in range(_NBUF)],
            [pltpu.SemaphoreType.DMA for _ in range(_NBUF)],
        ],
    )
    def gather_kernel(idx_hbm, table_hbm, t1_hbm, out_hbm, idx_v,
                      bufs0, bufs1, tails, sems0, sems1, ssems):
        wid = lax.axis_index("s") * _NC + lax.axis_index("c")
        base = wid * b_per_w
        t0 = table_hbm.at[:, pl.ds(0, _D0)]

        pltpu.sync_copy(idx_hbm.at[pl.ds(base * _SP, b_per_w * _SP)], idx_v)

        def gather_start(i, k):
            ids = idx_v.at[pl.ds(i * _SP, _S)]
            pltpu.async_copy(t0.at[ids], bufs0[k], sems0[k])
            pltpu.async_copy(t1_hbm.at[ids], bufs1[k], sems1[k])

        def gather_wait(k):
            ids = idx_v.at[pl.ds(0, _S)]
            pltpu.make_async_copy(t0.at[ids], bufs0[k], sems0[k]).wait()
            pltpu.make_async_copy(t1_hbm.at[ids], bufs1[k], sems1[k]).wait()

        def store_wait(k):
            pltpu.make_async_copy(
                bufs0[k], out_hbm.at[base, :, pl.ds(0, _D0)], ssems[k]).wait()
            pltpu.make_async_copy(
                tails[k], out_hbm.at[base, :, pl.ds(_D0, _D1)],
                ssems[k]).wait()

        for k in range(2):
            gather_start(k, k)

        def group(g, carry):
            for j in range(_NBUF):
                i = g * _NBUF + j
                b = base + i
                gather_wait(j)
                pltpu.async_copy(
                    bufs0[j], out_hbm.at[b, :, pl.ds(0, _D0)], ssems[j])
                for r in range(_S):
                    for o in (0, 16, 32, 48, 56):
                        tails[j][r, pl.ds(o, 16)] = bufs1[j][r, pl.ds(o, 16)]
                pltpu.async_copy(
                    tails[j], out_hbm.at[b, :, pl.ds(_D0, _D1)], ssems[j])
                k2 = (j + 2) % _NBUF

                @pl.when(i >= 2)
                def _():
                    store_wait(k2)

                nxt = i + 2

                @pl.when(nxt < b_per_w)
                def _():
                    gather_start(nxt, k2)
            return carry

        lax.fori_loop(0, b_per_w // _NBUF, group, 0, unroll=False)
        store_wait((b_per_w - 2) % _NBUF)
        store_wait((b_per_w - 1) % _NBUF)

    return gather_kernel


def kernel(input, table):
    nb = input.shape[0]
    idx1 = jnp.pad(input, ((0, 0), (0, _SP - _S))).reshape(-1)
    t1 = jnp.pad(table[:, _D0:], ((0, 0), (0, _D0 - _D1)))
    return _make_sc_gather(nb)(idx1, table, t1)
